# Initial kernel scaffold; baseline (speedup 1.0000x reference)
import jax
import jax.numpy as jnp
from jax.experimental import pallas as pl


def kernel(x, grid):
    raise NotImplementedError("write your pallas kernel here")


# speedup vs baseline: 1.3589x; 1.3589x over previous
import functools

import jax
import jax.numpy as jnp
from jax import lax
from jax.experimental import pallas as pl
from jax.experimental.pallas import tpu as pltpu
from jax.experimental.pallas import tpu_sc as plsc

N = 2097152
GX, GY, GZ = 512, 512, 128
LOWER_X, LOWER_Y, LOWER_Z = -4.0, -4.0, -1.0
RES = 64.0

NW = 32
SUBV = 128
LANES = 16


def _build(n_points, b_chunk):
  pw = n_points // NW
  b = min(b_chunk, pw)
  sub = b // SUBV
  nch = pw // b
  vps = SUBV // LANES
  assert pw % b == 0 and b % SUBV == 0

  mesh = plsc.VectorSubcoreMesh(core_axis_name="c", subcore_axis_name="s")

  scratch = (
      [pltpu.VMEM((b,), jnp.float32) for _ in range(3)]
      + [pltpu.VMEM((b,), jnp.int32) for _ in range(8)]
      + [pltpu.VMEM((b,), jnp.float32) for _ in range(8)]
      + [pltpu.VMEM((b,), jnp.float32) for _ in range(4)]
      + [pltpu.VMEM((b,), jnp.float32)]
      + [pltpu.SemaphoreType.DMA]
  )

  @functools.partial(
      pl.kernel,
      out_type=jax.ShapeDtypeStruct((n_points,), jnp.float32),
      mesh=mesh,
      scratch_types=scratch,
  )
  def vox(xs_hbm, ys_hbm, zs_hbm, g_hbm, out_hbm,
          px, py, pz,
          i000, i100, i010, i110, i001, i101, i011, i111,
          d000, d100, d010, d110, d001, d101, d011, d111,
          wfx, wfy, wfz, wvm, ob, sem):
    idx = (i000, i100, i010, i110, i001, i101, i011, i111)
    dst = (d000, d100, d010, d110, d001, d101, d011, d111)
    wid = lax.axis_index("s") * 2 + lax.axis_index("c")
    base0 = wid * pw

    def chunk(ci, carry):
      base = base0 + ci * b
      pltpu.sync_copy(xs_hbm.at[pl.ds(base, b)], px)
      pltpu.sync_copy(ys_hbm.at[pl.ds(base, b)], py)
      pltpu.sync_copy(zs_hbm.at[pl.ds(base, b)], pz)

      def comp(j, c2):
        for t in range(vps):
          s = pl.ds(j * SUBV + t * LANES, LANES)
          gx = (px[s] - LOWER_X) * RES
          gy = (py[s] - LOWER_Y) * RES
          gz = (pz[s] - LOWER_Z) * RES
          i0x = jnp.clip(gx.astype(jnp.int32), 0, GX - 1)
          i0y = jnp.clip(gy.astype(jnp.int32), 0, GY - 1)
          i0z = jnp.clip(gz.astype(jnp.int32), 0, GZ - 1)
          i1z = jnp.minimum(i0z + 1, GZ - 1)
          valid = ((gx >= 0.0) & (gx <= GX - 1.0)
                   & (gy >= 0.0) & (gy <= GY - 1.0)
                   & (gz >= 0.0) & (gz <= GZ - 1.0))
          lx0 = i0x * (GY * GZ)
          lx1 = jnp.minimum(i0x + 1, GX - 1) * (GY * GZ)
          ly0 = i0y * GZ
          ly1 = jnp.minimum(i0y + 1, GY - 1) * GZ
          b00 = lx0 + ly0
          b10 = lx1 + ly0
          b01 = lx0 + ly1
          b11 = lx1 + ly1
          i000[s] = b00 + i0z
          i100[s] = b10 + i0z
          i010[s] = b01 + i0z
          i110[s] = b11 + i0z
          i001[s] = b00 + i1z
          i101[s] = b10 + i1z
          i011[s] = b01 + i1z
          i111[s] = b11 + i1z
          wfx[s] = gx - i0x.astype(jnp.float32)
          wfy[s] = gy - i0y.astype(jnp.float32)
          wfz[s] = gz - i0z.astype(jnp.float32)
          wvm[s] = jnp.where(valid, 1.0, 0.0)
        sj = pl.ds(j * SUBV, SUBV)
        for c in range(8):
          pltpu.async_copy(g_hbm.at[idx[c].at[sj]], dst[c].at[sj], sem)
        return c2

      lax.fori_loop(0, sub, comp, 0)

      def interp(j, c2):
        sj = pl.ds(j * SUBV, SUBV)
        for c in range(8):
          pltpu.make_async_copy(g_hbm.at[idx[c].at[sj]],
                                dst[c].at[sj], sem).wait()
        for t in range(vps):
          s = pl.ds(j * SUBV + t * LANES, LANES)
          fx = wfx[s]
          fy = wfy[s]
          fz = wfz[s]
          vm = wvm[s]
          c00 = d000[s] + fx * (d100[s] - d000[s])
          c10 = d010[s] + fx * (d110[s] - d010[s])
          c01 = d001[s] + fx * (d101[s] - d001[s])
          c11 = d011[s] + fx * (d111[s] - d011[s])
          c0 = c00 + fy * (c10 - c00)
          c1 = c01 + fy * (c11 - c01)
          ob[s] = (c0 + fz * (c1 - c0)) * vm
        return c2

      lax.fori_loop(0, sub, interp, 0)
      pltpu.sync_copy(ob, out_hbm.at[pl.ds(base, b)])
      return carry

    lax.fori_loop(0, nch, chunk, 0)

  return vox


_VOX = _build(N, 1024)


def kernel(x, grid):
  xs = x[:, 0]
  ys = x[:, 1]
  zs = x[:, 2]
  g = grid.reshape(-1)
  sigma = _VOX(xs, ys, zs, g)
  alpha = jnp.zeros((N,), jnp.float32)
  return sigma, alpha


# speedup vs baseline: 1.6588x; 1.2207x over previous
import functools

import jax
import jax.numpy as jnp
from jax import lax
from jax.experimental import pallas as pl
from jax.experimental.pallas import tpu as pltpu
from jax.experimental.pallas import tpu_sc as plsc

N = 2097152
GX, GY, GZ = 512, 512, 128
LOWER_X, LOWER_Y, LOWER_Z = -4.0, -4.0, -1.0
RES = 64.0

NW = 32
SUBV = 128
LANES = 16


def _build(n_points, b_chunk):
  pw = n_points // NW
  b = min(b_chunk, pw)
  sub = b // SUBV
  nch = pw // b
  vps = SUBV // LANES
  assert pw % b == 0 and b % SUBV == 0 and nch % 2 == 0

  mesh = plsc.VectorSubcoreMesh(core_axis_name="c", subcore_axis_name="s")

  def one_set():
    return (
        [pltpu.VMEM((b,), jnp.float32) for _ in range(3)]
        + [pltpu.VMEM((b,), jnp.int32) for _ in range(8)]
        + [pltpu.VMEM((b,), jnp.float32) for _ in range(8)]
        + [pltpu.VMEM((b,), jnp.float32) for _ in range(4)]
        + [pltpu.VMEM((b,), jnp.float32)]
        + [pltpu.SemaphoreType.DMA]
    )

  @functools.partial(
      pl.kernel,
      out_type=jax.ShapeDtypeStruct((n_points,), jnp.float32),
      mesh=mesh,
      scratch_types=one_set() + one_set(),
  )
  def vox(xs_hbm, ys_hbm, zs_hbm, g_hbm, out_hbm, *refs):
    sets = (refs[:25], refs[25:])
    wid = lax.axis_index("s") * 2 + lax.axis_index("c")
    base0 = wid * pw

    def load_comp_fire(ci, st):
      (px, py, pz,
       i000, i100, i010, i110, i001, i101, i011, i111,
       _d0, _d1, _d2, _d3, _d4, _d5, _d6, _d7,
       wfx, wfy, wfz, wvm, _ob, sem) = st
      idx = (i000, i100, i010, i110, i001, i101, i011, i111)
      base = base0 + ci * b
      pltpu.sync_copy(xs_hbm.at[pl.ds(base, b)], px)
      pltpu.sync_copy(ys_hbm.at[pl.ds(base, b)], py)
      pltpu.sync_copy(zs_hbm.at[pl.ds(base, b)], pz)

      def comp(j, c2):
        for t in range(vps):
          s = pl.ds(j * SUBV + t * LANES, LANES)
          gx = (px[s] - LOWER_X) * RES
          gy = (py[s] - LOWER_Y) * RES
          gz = (pz[s] - LOWER_Z) * RES
          i0x = jnp.clip(gx.astype(jnp.int32), 0, GX - 1)
          i0y = jnp.clip(gy.astype(jnp.int32), 0, GY - 1)
          i0z = jnp.clip(gz.astype(jnp.int32), 0, GZ - 1)
          i1z = jnp.minimum(i0z + 1, GZ - 1)
          valid = ((gx >= 0.0) & (gx <= GX - 1.0)
                   & (gy >= 0.0) & (gy <= GY - 1.0)
                   & (gz >= 0.0) & (gz <= GZ - 1.0))
          lx0 = i0x * (GY * GZ)
          lx1 = jnp.minimum(i0x + 1, GX - 1) * (GY * GZ)
          ly0 = i0y * GZ
          ly1 = jnp.minimum(i0y + 1, GY - 1) * GZ
          b00 = lx0 + ly0
          b10 = lx1 + ly0
          b01 = lx0 + ly1
          b11 = lx1 + ly1
          i000[s] = b00 + i0z
          i100[s] = b10 + i0z
          i010[s] = b01 + i0z
          i110[s] = b11 + i0z
          i001[s] = b00 + i1z
          i101[s] = b10 + i1z
          i011[s] = b01 + i1z
          i111[s] = b11 + i1z
          wfx[s] = gx - i0x.astype(jnp.float32)
          wfy[s] = gy - i0y.astype(jnp.float32)
          wfz[s] = gz - i0z.astype(jnp.float32)
          wvm[s] = jnp.where(valid, 1.0, 0.0)
        sj = pl.ds(j * SUBV, SUBV)
        for c in range(8):
          pltpu.async_copy(g_hbm.at[idx[c].at[sj]], st[11 + c].at[sj], sem)
        return c2

      lax.fori_loop(0, sub, comp, 0)

    def drain_interp_store(ci, st):
      (_px, _py, _pz,
       i000, i100, i010, i110, i001, i101, i011, i111,
       d000, d100, d010, d110, d001, d101, d011, d111,
       wfx, wfy, wfz, wvm, ob, sem) = st
      idx = (i000, i100, i010, i110, i001, i101, i011, i111)
      base = base0 + ci * b

      def interp(j, c2):
        sj = pl.ds(j * SUBV, SUBV)
        for c in range(8):
          pltpu.make_async_copy(g_hbm.at[idx[c].at[sj]],
                                st[11 + c].at[sj], sem).wait()
        for t in range(vps):
          s = pl.ds(j * SUBV + t * LANES, LANES)
          fx = wfx[s]
          fy = wfy[s]
          fz = wfz[s]
          vm = wvm[s]
          c00 = d000[s] + fx * (d100[s] - d000[s])
          c10 = d010[s] + fx * (d110[s] - d010[s])
          c01 = d001[s] + fx * (d101[s] - d001[s])
          c11 = d011[s] + fx * (d111[s] - d011[s])
          c0 = c00 + fy * (c10 - c00)
          c1 = c01 + fy * (c11 - c01)
          ob[s] = (c0 + fz * (c1 - c0)) * vm
        return c2

      lax.fori_loop(0, sub, interp, 0)
      pltpu.sync_copy(ob, out_hbm.at[pl.ds(base, b)])

    load_comp_fire(0, sets[0])

    def pair(k, carry):
      ci = 2 * k
      load_comp_fire(ci + 1, sets[1])
      drain_interp_store(ci, sets[0])

      @pl.when(ci + 2 < nch)
      def _():
        load_comp_fire(ci + 2, sets[0])

      drain_interp_store(ci + 1, sets[1])
      return carry

    lax.fori_loop(0, nch // 2, pair, 0)

  return vox


_VOX = _build(N, 1024)


def kernel(x, grid):
  xs = x[:, 0]
  ys = x[:, 1]
  zs = x[:, 2]
  g = grid.reshape(-1)
  sigma = _VOX(xs, ys, zs, g)
  alpha = jnp.zeros((N,), jnp.float32)
  return sigma, alpha


# speedup vs baseline: 1.6951x; 1.0219x over previous
import functools

import jax
import jax.numpy as jnp
from jax import lax
from jax.experimental import pallas as pl
from jax.experimental.pallas import tpu as pltpu
from jax.experimental.pallas import tpu_sc as plsc

N = 2097152
GX, GY, GZ = 512, 512, 128
LOWER_X, LOWER_Y, LOWER_Z = -4.0, -4.0, -1.0
RES = 64.0

NW = 32
SUBV = 128
LANES = 16


def _build(n_points, b_chunk):
  pw = n_points // NW
  b = min(b_chunk, pw)
  sub = b // SUBV
  nch = pw // b
  vps = SUBV // LANES
  assert pw % b == 0 and b % SUBV == 0 and nch % 2 == 0

  mesh = plsc.VectorSubcoreMesh(core_axis_name="c", subcore_axis_name="s")

  def one_set():
    return (
        [pltpu.VMEM((b,), jnp.float32) for _ in range(3)]
        + [pltpu.VMEM((b,), jnp.int32) for _ in range(8)]
        + [pltpu.VMEM((b,), jnp.float32) for _ in range(8)]
        + [pltpu.VMEM((b,), jnp.float32) for _ in range(4)]
        + [pltpu.VMEM((b,), jnp.float32)]
        + [pltpu.SemaphoreType.DMA]
    )

  @functools.partial(
      pl.kernel,
      out_type=jax.ShapeDtypeStruct((n_points,), jnp.float32),
      mesh=mesh,
      scratch_types=one_set() + one_set(),
  )
  def vox(xs_hbm, ys_hbm, zs_hbm, g_hbm, out_hbm, *refs):
    sets = (refs[:25], refs[25:])
    wid = lax.axis_index("s") * 2 + lax.axis_index("c")
    base0 = wid * pw

    def load_comp_fire(ci, st):
      (px, py, pz,
       i000, i100, i010, i110, i001, i101, i011, i111,
       _d0, _d1, _d2, _d3, _d4, _d5, _d6, _d7,
       wfx, wfy, wfz, wvm, _ob, sem) = st
      idx = (i000, i100, i010, i110, i001, i101, i011, i111)
      base = base0 + ci * b
      pltpu.sync_copy(xs_hbm.at[pl.ds(base, b)], px)
      pltpu.sync_copy(ys_hbm.at[pl.ds(base, b)], py)
      pltpu.sync_copy(zs_hbm.at[pl.ds(base, b)], pz)

      def comp(j, c2):
        for t in range(vps):
          s = pl.ds(j * SUBV + t * LANES, LANES)
          gx = (px[s] - LOWER_X) * RES
          gy = (py[s] - LOWER_Y) * RES
          gz = (pz[s] - LOWER_Z) * RES
          i0x = jnp.clip(gx.astype(jnp.int32), 0, GX - 1)
          i0y = jnp.clip(gy.astype(jnp.int32), 0, GY - 1)
          i0z = jnp.clip(gz.astype(jnp.int32), 0, GZ - 1)
          i1z = jnp.minimum(i0z + 1, GZ - 1)
          valid = ((gx >= 0.0) & (gx <= GX - 1.0)
                   & (gy >= 0.0) & (gy <= GY - 1.0)
                   & (gz >= 0.0) & (gz <= GZ - 1.0))
          lx0 = i0x * (GY * GZ)
          lx1 = jnp.minimum(i0x + 1, GX - 1) * (GY * GZ)
          ly0 = i0y * GZ
          ly1 = jnp.minimum(i0y + 1, GY - 1) * GZ
          b00 = lx0 + ly0
          b10 = lx1 + ly0
          b01 = lx0 + ly1
          b11 = lx1 + ly1
          i000[s] = b00 + i0z
          i100[s] = b10 + i0z
          i010[s] = b01 + i0z
          i110[s] = b11 + i0z
          i001[s] = b00 + i1z
          i101[s] = b10 + i1z
          i011[s] = b01 + i1z
          i111[s] = b11 + i1z
          wfx[s] = gx - i0x.astype(jnp.float32)
          wfy[s] = gy - i0y.astype(jnp.float32)
          wfz[s] = gz - i0z.astype(jnp.float32)
          wvm[s] = jnp.where(valid, 1.0, 0.0)
        sj = pl.ds(j * SUBV, SUBV)
        for c in range(8):
          pltpu.async_copy(g_hbm.at[idx[c].at[sj]], st[11 + c].at[sj], sem)
        return c2

      lax.fori_loop(0, sub, comp, 0)

    def drain_interp_store(ci, st):
      (_px, _py, _pz,
       i000, i100, i010, i110, i001, i101, i011, i111,
       d000, d100, d010, d110, d001, d101, d011, d111,
       wfx, wfy, wfz, wvm, ob, sem) = st
      idx = (i000, i100, i010, i110, i001, i101, i011, i111)
      base = base0 + ci * b

      def interp(j, c2):
        sj = pl.ds(j * SUBV, SUBV)
        for c in range(8):
          pltpu.make_async_copy(g_hbm.at[idx[c].at[sj]],
                                st[11 + c].at[sj], sem).wait()
        for t in range(vps):
          s = pl.ds(j * SUBV + t * LANES, LANES)
          fx = wfx[s]
          fy = wfy[s]
          fz = wfz[s]
          vm = wvm[s]
          c00 = d000[s] + fx * (d100[s] - d000[s])
          c10 = d010[s] + fx * (d110[s] - d010[s])
          c01 = d001[s] + fx * (d101[s] - d001[s])
          c11 = d011[s] + fx * (d111[s] - d011[s])
          c0 = c00 + fy * (c10 - c00)
          c1 = c01 + fy * (c11 - c01)
          ob[s] = (c0 + fz * (c1 - c0)) * vm
        return c2

      lax.fori_loop(0, sub, interp, 0)
      pltpu.sync_copy(ob, out_hbm.at[pl.ds(base, b)])

    load_comp_fire(0, sets[0])

    def pair(k, carry):
      ci = 2 * k
      load_comp_fire(ci + 1, sets[1])
      drain_interp_store(ci, sets[0])

      @pl.when(ci + 2 < nch)
      def _():
        load_comp_fire(ci + 2, sets[0])

      drain_interp_store(ci + 1, sets[1])
      return carry

    lax.fori_loop(0, nch // 2, pair, 0)

  return vox


_VOX = _build(N, 2048)


def kernel(x, grid):
  xs = x[:, 0]
  ys = x[:, 1]
  zs = x[:, 2]
  g = grid.reshape(-1)
  sigma = _VOX(xs, ys, zs, g)
  alpha = jnp.zeros((N,), jnp.float32)
  return sigma, alpha
